# baseline (speedup 1.0000x reference)
import functools

import jax
import jax.numpy as jnp
from jax import lax
from jax.experimental import pallas as pl
from jax.experimental.pallas import tpu as pltpu
from jax.experimental.pallas import tpu_sc as plsc

N, D, DIN, DFF, E, NT, TH = 4096, 768, 1536, 512, 64, 3, 128
CAP = 80
EC = 256
ROWW = 384
BUF_ROWS = E * EC
BN = 512
NB = N // BN
BT = 512
NBT = N // BT
NC, NS = 2, 16
NW = NC * NS
HP = lax.Precision.HIGHEST
F32 = jnp.float32
DP = D // 2
_HI_MASK = -65536


def _bf16_bits_high(x):
    b = jax.lax.bitcast_convert_type(x, jnp.int32)
    lsb = jnp.right_shift(b, 16) & 1
    return (b + 0x7FFF + lsb) & _HI_MASK


def _pack_rows(x):
    lo = jnp.right_shift(_bf16_bits_high(x[:, :DP]), 16) & 0xFFFF
    hi = _bf16_bits_high(x[:, DP:])
    return jax.lax.bitcast_convert_type(hi | lo, F32)


def _unpack_rows(p):
    w = jax.lax.bitcast_convert_type(p, jnp.int32)
    lo = jax.lax.bitcast_convert_type(jnp.left_shift(w, 16), F32)
    hi = jax.lax.bitcast_convert_type(w & _HI_MASK, F32)
    return jnp.concatenate([lo, hi], axis=1)


def _bottom_body(x_ref, w1_ref, b1_ref, w2_ref, b2_ref, gw_ref,
                 h_ref, dst_ref, aux_ref, cnt_ref, imp_ref):
    i = pl.program_id(0)

    @pl.when(i == 0)
    def _():
        cnt_ref[...] = jnp.zeros_like(cnt_ref)
        imp_ref[...] = jnp.zeros_like(imp_ref)

    xb = x_ref[...]
    h1 = jnp.maximum(
        jnp.dot(xb.astype(jnp.bfloat16), w1_ref[...].astype(jnp.bfloat16),
                preferred_element_type=F32) + b1_ref[...], 0.0)
    h = jnp.dot(h1.astype(jnp.bfloat16), w2_ref[...].astype(jnp.bfloat16),
                preferred_element_type=F32) + b2_ref[...]
    lg = jnp.dot(h.astype(jnp.bfloat16), gw_ref[...].astype(jnp.bfloat16),
                 preferred_element_type=F32)

    iota_e = lax.broadcasted_iota(jnp.int32, (1, E), 1)
    r = lax.broadcasted_iota(jnp.int32, (BN, BN), 0)
    c = lax.broadcasted_iota(jnp.int32, (BN, BN), 1)
    ltri = (r > c).astype(F32)

    dcols, icols = [], []
    for t in range(NT):
        l = lg[:, t * E:(t + 1) * E]
        m = jnp.max(l, axis=1, keepdims=True)
        ex = jnp.exp(l - m)
        s = jnp.sum(ex, axis=1, keepdims=True)
        icols.append(jnp.sum(ex / s, axis=0, keepdims=True))
        cand = jnp.where(l == m, iota_e, E)
        am = jnp.min(cand, axis=1, keepdims=True)
        oh = (am == iota_e).astype(F32)
        prior = jnp.dot(ltri.astype(jnp.bfloat16), oh.astype(jnp.bfloat16),
                        preferred_element_type=F32)
        cnt = cnt_ref[t:t + 1, :]
        posf = jnp.sum(oh * (prior + cnt), axis=1, keepdims=True)
        cnt_ref[t:t + 1, :] = cnt + jnp.sum(oh, axis=0, keepdims=True)
        keep = posf < CAP
        posc = jnp.minimum(posf, CAP - 1).astype(jnp.int32)
        dst = am * EC + t * CAP + posc
        dcols.append(jnp.where(keep, dst, am * EC + NT * CAP + t))
    dst_ref[...] = jnp.concatenate(
        dcols + [jnp.zeros((BN, 128 - NT), jnp.int32)], axis=1)
    h_ref[...] = _pack_rows(h)
    impv = jnp.concatenate(icols, axis=1)
    imp_ref[...] = imp_ref[...] + jnp.broadcast_to(impv, imp_ref.shape)

    auxv = 0.0
    for tt in range(NT):
        imp = imp_ref[0:1, tt * E:(tt + 1) * E]
        mean = jnp.sum(imp) / E
        var = jnp.sum((imp - mean) ** 2) / E
        auxv = auxv + var / (mean * mean + 1e-9)
    aux_ref[...] = jnp.full((8, 128), auxv / NT, F32)


def _bottom(x, fc1_w, fc1_b, fc2_w, fc2_b, gw2):
    return pl.pallas_call(
        _bottom_body,
        grid=(NB,),
        in_specs=[
            pl.BlockSpec((BN, DIN), lambda i: (i, 0)),
            pl.BlockSpec((DIN, D), lambda i: (0, 0)),
            pl.BlockSpec((1, D), lambda i: (0, 0)),
            pl.BlockSpec((D, D), lambda i: (0, 0)),
            pl.BlockSpec((1, D), lambda i: (0, 0)),
            pl.BlockSpec((D, NT * E), lambda i: (0, 0)),
        ],
        out_specs=[
            pl.BlockSpec((BN, ROWW), lambda i: (i, 0)),
            pl.BlockSpec((BN, 128), lambda i: (i, 0)),
            pl.BlockSpec((8, 128), lambda i: (0, 0)),
        ],
        out_shape=[
            jax.ShapeDtypeStruct((N, ROWW), F32),
            jax.ShapeDtypeStruct((N, 128), jnp.int32),
            jax.ShapeDtypeStruct((8, 128), F32),
        ],
        scratch_shapes=[pltpu.VMEM((8, E), F32),
                        pltpu.VMEM((8, NT * E), F32)],
    )(x, fc1_w, fc1_b.reshape(1, D), fc2_w, fc2_b.reshape(1, D), gw2)


def _sc_scatter(h, dstf):
    mesh = plsc.VectorSubcoreMesh(core_axis_name="c", subcore_axis_name="s")
    scw = N // NW

    @functools.partial(
        pl.kernel, mesh=mesh,
        out_type=jax.ShapeDtypeStruct((BUF_ROWS, ROWW), F32),
        scratch_types=[
            pltpu.VMEM((scw,), jnp.int32),
            pltpu.VMEM((scw,), jnp.int32),
            pltpu.VMEM((scw,), jnp.int32),
            pltpu.VMEM((scw, ROWW), F32),
            pltpu.SemaphoreType.DMA,
            pltpu.SemaphoreType.DMA,
        ],
    )
    def k(h_hbm, idx_hbm, buf_hbm, idx0, idx1, idx2, rows_v, sem_i, sem_w):
        wid = lax.axis_index("s") * NC + lax.axis_index("c")
        base = wid * scw
        ch = pltpu.make_async_copy(h_hbm.at[pl.ds(base, scw)], rows_v, sem_i)
        ch.start()
        idxs = (idx0, idx1, idx2)
        loads = []
        for t in range(NT):
            c = pltpu.make_async_copy(
                idx_hbm.at[pl.ds(t * N + base, scw)], idxs[t], sem_i)
            c.start()
            loads.append(c)
        ch.wait()
        for c in loads:
            c.wait()
        writes = []
        for t in range(NT):
            c = pltpu.make_async_copy(rows_v, buf_hbm.at[idxs[t]], sem_w)
            c.start()
            writes.append(c)
        for c in writes:
            c.wait()

    return k(h, dstf)


def _ffn_body(buf_ref, gw_ref, w1_ref, b1_ref, w2_ref, b2_ref,
              tw1_ref, tb1_ref, tw2_ref, tb2_ref, q_ref):
    b = _unpack_rows(buf_ref[...])
    b = jnp.where(b != b, 0.0, b)
    b = jnp.clip(b, -1e30, 1e30)
    bb = b.astype(jnp.bfloat16)
    hid = jnp.dot(bb, w1_ref[0], preferred_element_type=F32) + b1_ref[0]
    hid = jnp.maximum(hid, 0.0)
    out = jnp.dot(hid.astype(jnp.bfloat16), w2_ref[0],
                  preferred_element_type=F32) + b2_ref[0]
    lg = jnp.dot(bb, gw_ref[...], preferred_element_type=F32)
    qsegs = []
    for t in range(NT):
        lseg = lg[t * CAP:(t + 1) * CAP, t * E:(t + 1) * E]
        m = jnp.max(lseg, axis=1, keepdims=True)
        sc = 1.0 / jnp.sum(jnp.exp(lseg - m), axis=1, keepdims=True)
        z = (out[t * CAP:(t + 1) * CAP, :] * sc).astype(jnp.bfloat16)
        th = jnp.maximum(
            jnp.dot(z, tw1_ref[t], preferred_element_type=F32)
            + tb1_ref[t], 0.0)
        b2s = jnp.max(tb2_ref[t], axis=1, keepdims=True)
        qsegs.append(jnp.sum(th * tw2_ref[t], axis=1, keepdims=True) + b2s)
    for t in range(NT):
        th0 = jnp.maximum(tb1_ref[t], 0.0)
        b2s = jnp.max(tb2_ref[t], axis=1, keepdims=True)
        qsegs.append(jnp.sum(th0 * tw2_ref[t], axis=1, keepdims=True) + b2s)
    qsegs.append(jnp.zeros((EC - NT * CAP - NT, 1), F32))
    e = pl.program_id(0)
    q_ref[pl.ds(2 * e, 2), :] = jnp.concatenate(qsegs, axis=0).reshape(2, 128)


def _ffn(buf, gwb, ew1, eb1, ew2, eb2, tw1, tb1, tw2r, tb2b):
    return pl.pallas_call(
        _ffn_body,
        grid=(E,),
        in_specs=[
            pl.BlockSpec((EC, ROWW), lambda e: (e, 0)),
            pl.BlockSpec((D, NT * E), lambda e: (0, 0)),
            pl.BlockSpec((1, D, DFF), lambda e: (e, 0, 0)),
            pl.BlockSpec((1, 1, DFF), lambda e: (e, 0, 0)),
            pl.BlockSpec((1, DFF, D), lambda e: (e, 0, 0)),
            pl.BlockSpec((1, 1, D), lambda e: (e, 0, 0)),
            pl.BlockSpec((NT, D, TH), lambda e: (0, 0, 0)),
            pl.BlockSpec((NT, 1, TH), lambda e: (0, 0, 0)),
            pl.BlockSpec((NT, 1, TH), lambda e: (0, 0, 0)),
            pl.BlockSpec((NT, 1, TH), lambda e: (0, 0, 0)),
        ],
        out_specs=pl.BlockSpec((BUF_ROWS // 128, 128), lambda e: (0, 0)),
        out_shape=jax.ShapeDtypeStruct((BUF_ROWS // 128, 128), F32),
    )(buf, gwb, ew1, eb1, ew2, eb2, tw1, tb1, tw2r, tb2b)


def _combine_body(dst_ref, q_ref, logits_ref):
    m = q_ref[...]
    iota128 = lax.broadcasted_iota(jnp.int32, (1, 128), 1)
    cols = []
    for t in range(NT):
        dcol = dst_ref[:, t:t + 1]
        hi = dcol // 128
        lo = dcol - hi * 128
        oh_hi = (hi == iota128).astype(F32)
        r = jnp.dot(oh_hi, m, precision=HP)
        oh_lo = (lo == iota128).astype(F32)
        cols.append(jnp.sum(r * oh_lo, axis=1, keepdims=True))
    logits_ref[...] = jnp.concatenate(
        cols + [jnp.zeros((BN, 128 - NT), F32)], axis=1)


def _combine(dstq, q):
    return pl.pallas_call(
        _combine_body,
        grid=(NB,),
        in_specs=[
            pl.BlockSpec((BN, 128), lambda i: (i, 0)),
            pl.BlockSpec((BUF_ROWS // 128, 128), lambda i: (0, 0)),
        ],
        out_specs=pl.BlockSpec((BN, 128), lambda i: (i, 0)),
        out_shape=jax.ShapeDtypeStruct((N, 128), F32),
    )(dstq, q)


def kernel(x, fc1_w, fc1_b, fc2_w, fc2_b, gate_w, expert_w1, expert_b1,
           expert_w2, expert_b2, tower_w1, tower_b1, tower_w2, tower_b2):
    gw2 = gate_w.transpose(1, 0, 2).reshape(D, NT * E)
    hrow, dstq, auxm = _bottom(x, fc1_w, fc1_b, fc2_w, fc2_b, gw2)
    dstf = dstq[:, :NT].T.reshape(NT * N)
    buf = _sc_scatter(hrow, dstf)
    q = _ffn(buf,
             gw2.astype(jnp.bfloat16),
             expert_w1.astype(jnp.bfloat16),
             expert_b1.reshape(E, 1, DFF),
             expert_w2.astype(jnp.bfloat16),
             expert_b2.reshape(E, 1, D),
             tower_w1.astype(jnp.bfloat16),
             tower_b1.reshape(NT, 1, TH),
             tower_w2.reshape(NT, 1, TH),
             jnp.broadcast_to(tower_b2.reshape(NT, 1, 1), (NT, 1, TH)))
    tlq = _combine(dstq, q)
    logits = tlq[:, :NT].T
    return logits, auxm[0, 0]


# speedup vs baseline: 1.1613x; 1.1613x over previous
import functools

import jax
import jax.numpy as jnp
from jax import lax
from jax.experimental import pallas as pl
from jax.experimental.pallas import tpu as pltpu
from jax.experimental.pallas import tpu_sc as plsc

N, D, DIN, DFF, E, NT, TH = 4096, 768, 1536, 512, 64, 3, 128
CAP = 80
EC = 256
ROWW = 512
BUF_ROWS = E * EC
BN = 512
NB = N // BN
BT = 512
NBT = N // BT
NC, NS = 2, 16
NW = NC * NS
HP = lax.Precision.HIGHEST
F32 = jnp.float32
DP = D // 2
_HI_MASK = -65536


def _bf16_bits_high(x):
    b = jax.lax.bitcast_convert_type(x, jnp.int32)
    lsb = jnp.right_shift(b, 16) & 1
    return (b + 0x7FFF + lsb) & _HI_MASK


def _pack_rows(x):
    lo = jnp.right_shift(_bf16_bits_high(x[:, :DP]), 16) & 0xFFFF
    hi = _bf16_bits_high(x[:, DP:])
    return jax.lax.bitcast_convert_type(hi | lo, F32)


def _unpack_rows(p):
    w = jax.lax.bitcast_convert_type(p, jnp.int32)
    lo = jax.lax.bitcast_convert_type(jnp.left_shift(w, 16), F32)
    hi = jax.lax.bitcast_convert_type(w & _HI_MASK, F32)
    return jnp.concatenate([lo, hi], axis=1)


def _bottom_body(x_ref, w1_ref, b1_ref, w2_ref, b2_ref, gw_ref,
                 h_ref, dst_ref, aux_ref, cnt_ref, imp_ref):
    i = pl.program_id(0)

    @pl.when(i == 0)
    def _():
        cnt_ref[...] = jnp.zeros_like(cnt_ref)
        imp_ref[...] = jnp.zeros_like(imp_ref)

    xb = x_ref[...]
    h1 = jnp.maximum(
        jnp.dot(xb.astype(jnp.bfloat16), w1_ref[...].astype(jnp.bfloat16),
                preferred_element_type=F32) + b1_ref[...], 0.0)
    h = jnp.dot(h1.astype(jnp.bfloat16), w2_ref[...].astype(jnp.bfloat16),
                preferred_element_type=F32) + b2_ref[...]
    lg = jnp.dot(h.astype(jnp.bfloat16), gw_ref[...].astype(jnp.bfloat16),
                 preferred_element_type=F32)

    iota_e = lax.broadcasted_iota(jnp.int32, (1, E), 1)
    r = lax.broadcasted_iota(jnp.int32, (BN, BN), 0)
    c = lax.broadcasted_iota(jnp.int32, (BN, BN), 1)
    ltri = (r > c).astype(F32)

    dcols, scols, icols = [], [], []
    for t in range(NT):
        l = lg[:, t * E:(t + 1) * E]
        m = jnp.max(l, axis=1, keepdims=True)
        ex = jnp.exp(l - m)
        s = jnp.sum(ex, axis=1, keepdims=True)
        gate = 1.0 / s
        icols.append(jnp.sum(ex / s, axis=0, keepdims=True))
        cand = jnp.where(l == m, iota_e, E)
        am = jnp.min(cand, axis=1, keepdims=True)
        oh = (am == iota_e).astype(F32)
        prior = jnp.dot(ltri.astype(jnp.bfloat16), oh.astype(jnp.bfloat16),
                        preferred_element_type=F32)
        cnt = cnt_ref[t:t + 1, :]
        posf = jnp.sum(oh * (prior + cnt), axis=1, keepdims=True)
        cnt_ref[t:t + 1, :] = cnt + jnp.sum(oh, axis=0, keepdims=True)
        keep = posf < CAP
        posc = jnp.minimum(posf, CAP - 1).astype(jnp.int32)
        dst = am * EC + t * CAP + posc
        dcols.append(jnp.where(keep, dst, am * EC + NT * CAP + t))
        scols.append(jnp.where(keep, gate, 0.0))
    dst_ref[...] = jnp.concatenate(
        dcols + [jnp.zeros((BN, 128 - NT), jnp.int32)], axis=1)
    h_ref[...] = jnp.concatenate(
        [_pack_rows(h)] + scols + [jnp.zeros((BN, ROWW - DP - NT), F32)],
        axis=1)
    impv = jnp.concatenate(icols, axis=1)
    imp_ref[...] = imp_ref[...] + jnp.broadcast_to(impv, imp_ref.shape)

    auxv = 0.0
    for tt in range(NT):
        imp = imp_ref[0:1, tt * E:(tt + 1) * E]
        mean = jnp.sum(imp) / E
        var = jnp.sum((imp - mean) ** 2) / E
        auxv = auxv + var / (mean * mean + 1e-9)
    aux_ref[...] = jnp.full((8, 128), auxv / NT, F32)


def _bottom(x, fc1_w, fc1_b, fc2_w, fc2_b, gw2):
    return pl.pallas_call(
        _bottom_body,
        grid=(NB,),
        in_specs=[
            pl.BlockSpec((BN, DIN), lambda i: (i, 0)),
            pl.BlockSpec((DIN, D), lambda i: (0, 0)),
            pl.BlockSpec((1, D), lambda i: (0, 0)),
            pl.BlockSpec((D, D), lambda i: (0, 0)),
            pl.BlockSpec((1, D), lambda i: (0, 0)),
            pl.BlockSpec((D, NT * E), lambda i: (0, 0)),
        ],
        out_specs=[
            pl.BlockSpec((BN, ROWW), lambda i: (i, 0)),
            pl.BlockSpec((BN, 128), lambda i: (i, 0)),
            pl.BlockSpec((8, 128), lambda i: (0, 0)),
        ],
        out_shape=[
            jax.ShapeDtypeStruct((N, ROWW), F32),
            jax.ShapeDtypeStruct((N, 128), jnp.int32),
            jax.ShapeDtypeStruct((8, 128), F32),
        ],
        scratch_shapes=[pltpu.VMEM((8, E), F32),
                        pltpu.VMEM((8, NT * E), F32)],
    )(x, fc1_w, fc1_b.reshape(1, D), fc2_w, fc2_b.reshape(1, D), gw2)


def _sc_scatter(h, dstf):
    mesh = plsc.VectorSubcoreMesh(core_axis_name="c", subcore_axis_name="s")
    scw = N // NW

    @functools.partial(
        pl.kernel, mesh=mesh,
        out_type=jax.ShapeDtypeStruct((BUF_ROWS, ROWW), F32),
        scratch_types=[
            pltpu.VMEM((scw,), jnp.int32),
            pltpu.VMEM((scw,), jnp.int32),
            pltpu.VMEM((scw,), jnp.int32),
            pltpu.VMEM((scw, ROWW), F32),
            pltpu.SemaphoreType.DMA,
            pltpu.SemaphoreType.DMA,
        ],
    )
    def k(h_hbm, idx_hbm, buf_hbm, idx0, idx1, idx2, rows_v, sem_i, sem_w):
        wid = lax.axis_index("s") * NC + lax.axis_index("c")
        base = wid * scw
        ch = pltpu.make_async_copy(h_hbm.at[pl.ds(base, scw)], rows_v, sem_i)
        ch.start()
        idxs = (idx0, idx1, idx2)
        loads = []
        for t in range(NT):
            c = pltpu.make_async_copy(
                idx_hbm.at[pl.ds(t * N + base, scw)], idxs[t], sem_i)
            c.start()
            loads.append(c)
        ch.wait()
        for c in loads:
            c.wait()
        writes = []
        for t in range(NT):
            c = pltpu.make_async_copy(rows_v, buf_hbm.at[idxs[t]], sem_w)
            c.start()
            writes.append(c)
        for c in writes:
            c.wait()

    return k(h, dstf)


def _ffn_body(buf_ref, w1_ref, b1_ref, w2_ref, b2_ref,
              tw1_ref, tb1_ref, tw2_ref, tb2_ref, q_ref):
    blk = buf_ref[...]
    b = _unpack_rows(blk[:, :DP])
    b = jnp.where(b != b, 0.0, b)
    b = jnp.clip(b, -1e30, 1e30)
    hid = jnp.dot(b.astype(jnp.bfloat16), w1_ref[0],
                  preferred_element_type=F32) + b1_ref[0]
    hid = jnp.maximum(hid, 0.0)
    out = jnp.dot(hid.astype(jnp.bfloat16), w2_ref[0].astype(jnp.bfloat16),
                  preferred_element_type=F32) + b2_ref[0]
    qsegs = []
    for t in range(NT):
        sc = blk[t * CAP:(t + 1) * CAP, DP + t:DP + t + 1]
        sc = jnp.where(sc != sc, 0.0, jnp.clip(sc, -1e30, 1e30))
        z = (out[t * CAP:(t + 1) * CAP, :] * sc).astype(jnp.bfloat16)
        th = jnp.maximum(
            jnp.dot(z, tw1_ref[t], preferred_element_type=F32)
            + tb1_ref[t], 0.0)
        b2s = jnp.max(tb2_ref[t], axis=1, keepdims=True)
        qsegs.append(jnp.sum(th * tw2_ref[t], axis=1, keepdims=True) + b2s)
    for t in range(NT):
        th0 = jnp.maximum(tb1_ref[t], 0.0)
        b2s = jnp.max(tb2_ref[t], axis=1, keepdims=True)
        qsegs.append(jnp.sum(th0 * tw2_ref[t], axis=1, keepdims=True) + b2s)
    qsegs.append(jnp.zeros((EC - NT * CAP - NT, 1), F32))
    e = pl.program_id(0)
    q_ref[pl.ds(2 * e, 2), :] = jnp.concatenate(qsegs, axis=0).reshape(2, 128)


def _ffn(buf, ew1, eb1, ew2, eb2, tw1, tb1, tw2r, tb2b):
    return pl.pallas_call(
        _ffn_body,
        grid=(E,),
        in_specs=[
            pl.BlockSpec((EC, ROWW), lambda e: (e, 0)),
            pl.BlockSpec((1, D, DFF), lambda e: (e, 0, 0)),
            pl.BlockSpec((1, 1, DFF), lambda e: (e, 0, 0)),
            pl.BlockSpec((1, DFF, D), lambda e: (e, 0, 0)),
            pl.BlockSpec((1, 1, D), lambda e: (e, 0, 0)),
            pl.BlockSpec((NT, D, TH), lambda e: (0, 0, 0)),
            pl.BlockSpec((NT, 1, TH), lambda e: (0, 0, 0)),
            pl.BlockSpec((NT, 1, TH), lambda e: (0, 0, 0)),
            pl.BlockSpec((NT, 1, TH), lambda e: (0, 0, 0)),
        ],
        out_specs=pl.BlockSpec((BUF_ROWS // 128, 128), lambda e: (0, 0)),
        out_shape=jax.ShapeDtypeStruct((BUF_ROWS // 128, 128), F32),
    )(buf, ew1, eb1, ew2, eb2, tw1, tb1, tw2r, tb2b)


def _combine_body(dst_ref, q_ref, logits_ref):
    m = q_ref[...]
    iota128 = lax.broadcasted_iota(jnp.int32, (1, 128), 1)
    cols = []
    for t in range(NT):
        dcol = dst_ref[:, t:t + 1]
        hi = dcol // 128
        lo = dcol - hi * 128
        oh_hi = (hi == iota128).astype(F32)
        r = jnp.dot(oh_hi, m, precision=HP)
        oh_lo = (lo == iota128).astype(F32)
        cols.append(jnp.sum(r * oh_lo, axis=1, keepdims=True))
    logits_ref[...] = jnp.concatenate(
        cols + [jnp.zeros((BN, 128 - NT), F32)], axis=1)


def _combine(dstq, q):
    return pl.pallas_call(
        _combine_body,
        grid=(NB,),
        in_specs=[
            pl.BlockSpec((BN, 128), lambda i: (i, 0)),
            pl.BlockSpec((BUF_ROWS // 128, 128), lambda i: (0, 0)),
        ],
        out_specs=pl.BlockSpec((BN, 128), lambda i: (i, 0)),
        out_shape=jax.ShapeDtypeStruct((N, 128), F32),
    )(dstq, q)


def kernel(x, fc1_w, fc1_b, fc2_w, fc2_b, gate_w, expert_w1, expert_b1,
           expert_w2, expert_b2, tower_w1, tower_b1, tower_w2, tower_b2):
    gw2 = gate_w.transpose(1, 0, 2).reshape(D, NT * E)
    hrow, dstq, auxm = _bottom(x, fc1_w, fc1_b, fc2_w, fc2_b, gw2)
    dstf = dstq[:, :NT].T.reshape(NT * N)
    buf = _sc_scatter(hrow, dstf)
    q = _ffn(buf,
             expert_w1.astype(jnp.bfloat16),
             expert_b1.reshape(E, 1, DFF),
             expert_w2,
             expert_b2.reshape(E, 1, D),
             tower_w1.astype(jnp.bfloat16),
             tower_b1.reshape(NT, 1, TH),
             tower_w2.reshape(NT, 1, TH),
             jnp.broadcast_to(tower_b2.reshape(NT, 1, 1), (NT, 1, TH)))
    tlq = _combine(dstq, q)
    logits = tlq[:, :NT].T
    return logits, auxm[0, 0]


# speedup vs baseline: 1.1908x; 1.0254x over previous
import functools

import jax
import jax.numpy as jnp
from jax import lax
from jax.experimental import pallas as pl
from jax.experimental.pallas import tpu as pltpu
from jax.experimental.pallas import tpu_sc as plsc

N, D, DIN, DFF, E, NT, TH = 4096, 768, 1536, 512, 64, 3, 128
CAP = 80
EC = 256
ROWW = 512
BUF_ROWS = E * EC
BN = 1024
NB = N // BN
BT = 512
NBT = N // BT
NC, NS = 2, 16
NW = NC * NS
HP = lax.Precision.HIGHEST
F32 = jnp.float32
DP = D // 2
_HI_MASK = -65536


def _bf16_bits_high(x):
    b = jax.lax.bitcast_convert_type(x, jnp.int32)
    lsb = jnp.right_shift(b, 16) & 1
    return (b + 0x7FFF + lsb) & _HI_MASK


def _pack_rows(x):
    lo = jnp.right_shift(_bf16_bits_high(x[:, :DP]), 16) & 0xFFFF
    hi = _bf16_bits_high(x[:, DP:])
    return jax.lax.bitcast_convert_type(hi | lo, F32)


def _unpack_rows(p):
    w = jax.lax.bitcast_convert_type(p, jnp.int32)
    lo = jax.lax.bitcast_convert_type(jnp.left_shift(w, 16), F32)
    hi = jax.lax.bitcast_convert_type(w & _HI_MASK, F32)
    return jnp.concatenate([lo, hi], axis=1)


def _bottom_body(x_ref, w1_ref, b1_ref, w2_ref, b2_ref, gw_ref,
                 h_ref, dst_ref, aux_ref, cnt_ref, imp_ref):
    i = pl.program_id(0)

    @pl.when(i == 0)
    def _():
        cnt_ref[...] = jnp.zeros_like(cnt_ref)
        imp_ref[...] = jnp.zeros_like(imp_ref)

    xb = x_ref[...]
    h1 = jnp.maximum(
        jnp.dot(xb.astype(jnp.bfloat16), w1_ref[...].astype(jnp.bfloat16),
                preferred_element_type=F32) + b1_ref[...], 0.0)
    h = jnp.dot(h1.astype(jnp.bfloat16), w2_ref[...].astype(jnp.bfloat16),
                preferred_element_type=F32) + b2_ref[...]
    lg = jnp.dot(h.astype(jnp.bfloat16), gw_ref[...].astype(jnp.bfloat16),
                 preferred_element_type=F32)

    iota_e = lax.broadcasted_iota(jnp.int32, (1, E), 1)
    r = lax.broadcasted_iota(jnp.int32, (BN, BN), 0)
    c = lax.broadcasted_iota(jnp.int32, (BN, BN), 1)
    ltri = (r > c).astype(F32)

    dcols, scols, icols = [], [], []
    for t in range(NT):
        l = lg[:, t * E:(t + 1) * E]
        m = jnp.max(l, axis=1, keepdims=True)
        ex = jnp.exp(l - m)
        s = jnp.sum(ex, axis=1, keepdims=True)
        gate = 1.0 / s
        icols.append(jnp.sum(ex / s, axis=0, keepdims=True))
        cand = jnp.where(l == m, iota_e, E)
        am = jnp.min(cand, axis=1, keepdims=True)
        oh = (am == iota_e).astype(F32)
        prior = jnp.dot(ltri.astype(jnp.bfloat16), oh.astype(jnp.bfloat16),
                        preferred_element_type=F32)
        cnt = cnt_ref[t:t + 1, :]
        posf = jnp.sum(oh * (prior + cnt), axis=1, keepdims=True)
        cnt_ref[t:t + 1, :] = cnt + jnp.sum(oh, axis=0, keepdims=True)
        keep = posf < CAP
        posc = jnp.minimum(posf, CAP - 1).astype(jnp.int32)
        dst = am * EC + t * CAP + posc
        dcols.append(jnp.where(keep, dst, am * EC + NT * CAP + t))
        scols.append(jnp.where(keep, gate, 0.0))
    dst_ref[...] = jnp.concatenate(
        dcols + [jnp.zeros((BN, 128 - NT), jnp.int32)], axis=1)
    h_ref[...] = jnp.concatenate(
        [_pack_rows(h)] + scols + [jnp.zeros((BN, ROWW - DP - NT), F32)],
        axis=1)
    impv = jnp.concatenate(icols, axis=1)
    imp_ref[...] = imp_ref[...] + jnp.broadcast_to(impv, imp_ref.shape)

    auxv = 0.0
    for tt in range(NT):
        imp = imp_ref[0:1, tt * E:(tt + 1) * E]
        mean = jnp.sum(imp) / E
        var = jnp.sum((imp - mean) ** 2) / E
        auxv = auxv + var / (mean * mean + 1e-9)
    aux_ref[...] = jnp.full((8, 128), auxv / NT, F32)


def _bottom(x, fc1_w, fc1_b, fc2_w, fc2_b, gw2):
    return pl.pallas_call(
        _bottom_body,
        grid=(NB,),
        in_specs=[
            pl.BlockSpec((BN, DIN), lambda i: (i, 0)),
            pl.BlockSpec((DIN, D), lambda i: (0, 0)),
            pl.BlockSpec((1, D), lambda i: (0, 0)),
            pl.BlockSpec((D, D), lambda i: (0, 0)),
            pl.BlockSpec((1, D), lambda i: (0, 0)),
            pl.BlockSpec((D, NT * E), lambda i: (0, 0)),
        ],
        out_specs=[
            pl.BlockSpec((BN, ROWW), lambda i: (i, 0)),
            pl.BlockSpec((BN, 128), lambda i: (i, 0)),
            pl.BlockSpec((8, 128), lambda i: (0, 0)),
        ],
        out_shape=[
            jax.ShapeDtypeStruct((N, ROWW), F32),
            jax.ShapeDtypeStruct((N, 128), jnp.int32),
            jax.ShapeDtypeStruct((8, 128), F32),
        ],
        scratch_shapes=[pltpu.VMEM((8, E), F32),
                        pltpu.VMEM((8, NT * E), F32)],
    )(x, fc1_w, fc1_b.reshape(1, D), fc2_w, fc2_b.reshape(1, D), gw2)


def _sc_scatter(h, dstf):
    mesh = plsc.VectorSubcoreMesh(core_axis_name="c", subcore_axis_name="s")
    scw = N // NW

    @functools.partial(
        pl.kernel, mesh=mesh,
        out_type=jax.ShapeDtypeStruct((BUF_ROWS, ROWW), F32),
        scratch_types=[
            pltpu.VMEM((scw,), jnp.int32),
            pltpu.VMEM((scw,), jnp.int32),
            pltpu.VMEM((scw,), jnp.int32),
            pltpu.VMEM((scw, ROWW), F32),
            pltpu.SemaphoreType.DMA,
            pltpu.SemaphoreType.DMA,
        ],
    )
    def k(h_hbm, idx_hbm, buf_hbm, idx0, idx1, idx2, rows_v, sem_i, sem_w):
        wid = lax.axis_index("s") * NC + lax.axis_index("c")
        base = wid * scw
        ch = pltpu.make_async_copy(h_hbm.at[pl.ds(base, scw)], rows_v, sem_i)
        ch.start()
        idxs = (idx0, idx1, idx2)
        loads = []
        for t in range(NT):
            c = pltpu.make_async_copy(
                idx_hbm.at[pl.ds(t * N + base, scw)], idxs[t], sem_i)
            c.start()
            loads.append(c)
        ch.wait()
        for c in loads:
            c.wait()
        writes = []
        for t in range(NT):
            c = pltpu.make_async_copy(rows_v, buf_hbm.at[idxs[t]], sem_w)
            c.start()
            writes.append(c)
        for c in writes:
            c.wait()

    return k(h, dstf)


def _ffn_body(buf_ref, w1_ref, b1_ref, w2_ref, b2_ref,
              tw1_ref, tb1_ref, tw2_ref, tb2_ref, q_ref):
    blk = buf_ref[...]
    b = _unpack_rows(blk[:, :DP])
    b = jnp.where(b != b, 0.0, b)
    b = jnp.clip(b, -1e30, 1e30)
    hid = jnp.dot(b.astype(jnp.bfloat16), w1_ref[0],
                  preferred_element_type=F32) + b1_ref[0]
    hid = jnp.maximum(hid, 0.0)
    out = jnp.dot(hid.astype(jnp.bfloat16), w2_ref[0].astype(jnp.bfloat16),
                  preferred_element_type=F32) + b2_ref[0]
    qsegs = []
    for t in range(NT):
        sc = blk[t * CAP:(t + 1) * CAP, DP + t:DP + t + 1]
        sc = jnp.where(sc != sc, 0.0, jnp.clip(sc, -1e30, 1e30))
        z = (out[t * CAP:(t + 1) * CAP, :] * sc).astype(jnp.bfloat16)
        th = jnp.maximum(
            jnp.dot(z, tw1_ref[t], preferred_element_type=F32)
            + tb1_ref[t], 0.0)
        b2s = jnp.max(tb2_ref[t], axis=1, keepdims=True)
        qsegs.append(jnp.sum(th * tw2_ref[t], axis=1, keepdims=True) + b2s)
    for t in range(NT):
        th0 = jnp.maximum(tb1_ref[t], 0.0)
        b2s = jnp.max(tb2_ref[t], axis=1, keepdims=True)
        qsegs.append(jnp.sum(th0 * tw2_ref[t], axis=1, keepdims=True) + b2s)
    qsegs.append(jnp.zeros((EC - NT * CAP - NT, 1), F32))
    e = pl.program_id(0)
    q_ref[pl.ds(2 * e, 2), :] = jnp.concatenate(qsegs, axis=0).reshape(2, 128)


def _ffn(buf, ew1, eb1, ew2, eb2, tw1, tb1, tw2r, tb2b):
    return pl.pallas_call(
        _ffn_body,
        grid=(E,),
        in_specs=[
            pl.BlockSpec((EC, ROWW), lambda e: (e, 0)),
            pl.BlockSpec((1, D, DFF), lambda e: (e, 0, 0)),
            pl.BlockSpec((1, 1, DFF), lambda e: (e, 0, 0)),
            pl.BlockSpec((1, DFF, D), lambda e: (e, 0, 0)),
            pl.BlockSpec((1, 1, D), lambda e: (e, 0, 0)),
            pl.BlockSpec((NT, D, TH), lambda e: (0, 0, 0)),
            pl.BlockSpec((NT, 1, TH), lambda e: (0, 0, 0)),
            pl.BlockSpec((NT, 1, TH), lambda e: (0, 0, 0)),
            pl.BlockSpec((NT, 1, TH), lambda e: (0, 0, 0)),
        ],
        out_specs=pl.BlockSpec((BUF_ROWS // 128, 128), lambda e: (0, 0)),
        out_shape=jax.ShapeDtypeStruct((BUF_ROWS // 128, 128), F32),
    )(buf, ew1, eb1, ew2, eb2, tw1, tb1, tw2r, tb2b)


def _combine_body(dst_ref, q_ref, logits_ref):
    m = q_ref[...]
    iota128 = lax.broadcasted_iota(jnp.int32, (1, 128), 1)
    cols = []
    for t in range(NT):
        dcol = dst_ref[:, t:t + 1]
        hi = dcol // 128
        lo = dcol - hi * 128
        oh_hi = (hi == iota128).astype(F32)
        r = jnp.dot(oh_hi, m, precision=HP)
        oh_lo = (lo == iota128).astype(F32)
        cols.append(jnp.sum(r * oh_lo, axis=1, keepdims=True))
    logits_ref[...] = jnp.concatenate(
        cols + [jnp.zeros((BN, 128 - NT), F32)], axis=1)


def _combine(dstq, q):
    return pl.pallas_call(
        _combine_body,
        grid=(NB,),
        in_specs=[
            pl.BlockSpec((BN, 128), lambda i: (i, 0)),
            pl.BlockSpec((BUF_ROWS // 128, 128), lambda i: (0, 0)),
        ],
        out_specs=pl.BlockSpec((BN, 128), lambda i: (i, 0)),
        out_shape=jax.ShapeDtypeStruct((N, 128), F32),
    )(dstq, q)


def kernel(x, fc1_w, fc1_b, fc2_w, fc2_b, gate_w, expert_w1, expert_b1,
           expert_w2, expert_b2, tower_w1, tower_b1, tower_w2, tower_b2):
    gw2 = gate_w.transpose(1, 0, 2).reshape(D, NT * E)
    hrow, dstq, auxm = _bottom(x, fc1_w, fc1_b, fc2_w, fc2_b, gw2)
    dstf = dstq[:, :NT].T.reshape(NT * N)
    buf = _sc_scatter(hrow, dstf)
    q = _ffn(buf,
             expert_w1.astype(jnp.bfloat16),
             expert_b1.reshape(E, 1, DFF),
             expert_w2,
             expert_b2.reshape(E, 1, D),
             tower_w1.astype(jnp.bfloat16),
             tower_b1.reshape(NT, 1, TH),
             tower_w2.reshape(NT, 1, TH),
             jnp.broadcast_to(tower_b2.reshape(NT, 1, 1), (NT, 1, TH)))
    tlq = _combine(dstq, q)
    logits = tlq[:, :NT].T
    return logits, auxm[0, 0]
